# Initial kernel scaffold; baseline (speedup 1.0000x reference)
import jax
import jax.numpy as jnp
from jax.experimental import pallas as pl


def kernel(x, params, edge_index, pred_edge_index):
    raise NotImplementedError("write your pallas kernel here")


# speedup vs baseline: 1.0212x; 1.0212x over previous
import functools

import jax
import jax.numpy as jnp
import numpy as np
from jax.experimental import pallas as pl
from jax.experimental.pallas import tpu as pltpu

N_NODES = 10000
D_IN = 128
HID = 768
OUT = 384

_BN_SCALE = 1.0 / np.sqrt(1.0 + 1e-5)


def _bn(x, g, b):
    return g * x * _BN_SCALE + b


def _gelu(x):
    return 0.5 * x * (1.0 + jax.lax.erf(x * np.float32(1.0 / np.sqrt(2.0))))


def _gat_conv(x, src, dst, W, a_src, a_dst, bias, heads, concat, n):
    c = W.shape[1] // heads
    h = (x @ W).reshape(n, heads, c)
    a_s = (h * a_src[None]).sum(-1)
    a_d = (h * a_dst[None]).sum(-1)
    alpha = jax.nn.leaky_relu(a_s[src] + a_d[dst], negative_slope=0.2)
    m = jax.ops.segment_max(alpha, dst, num_segments=n)
    m = jnp.where(jnp.isneginf(m), 0.0, m)
    e = jnp.exp(alpha - m[dst])
    denom = jax.ops.segment_sum(e, dst, num_segments=n)
    w = e / (denom[dst] + 1e-16)
    out = jax.ops.segment_sum(h[src] * w[:, :, None], dst, num_segments=n)
    if concat:
        out = out.reshape(n, heads * c)
    else:
        out = out.mean(axis=1)
    return out + bias


_HR = 2000


def _head_body(se_ref, de_ref,
               wq_ref, wk_ref, wv_ref, wo_ref, bq_ref, bk_ref, bv_ref, bo_ref,
               w1_ref, b1_ref, g1_ref, be1_ref,
               w2_ref, b2_ref, g2_ref, be2_ref,
               w3_ref, b3_ref, g3_ref, be3_ref,
               w4_ref, b4_ref, w5_ref, b5_ref,
               out_ref):
    se = se_ref[...]
    de = de_ref[...]

    def mm(a, w):
        return jax.lax.dot_general(a, w, (((1,), (0,)), ((), ())),
                                   preferred_element_type=jnp.float32)

    q1 = mm(se, wq_ref[...]) + bq_ref[...][None, :]
    q2 = mm(de, wq_ref[...]) + bq_ref[...][None, :]
    k1 = mm(se, wk_ref[...]) + bk_ref[...][None, :]
    k2 = mm(de, wk_ref[...]) + bk_ref[...][None, :]
    v1 = mm(se, wv_ref[...]) + bv_ref[...][None, :]
    v2 = mm(de, wv_ref[...]) + bv_ref[...][None, :]

    dh = 96
    scale = 1.0 / np.sqrt(dh)
    o_chunks1 = []
    o_chunks2 = []
    for hh in range(4):
        sl = slice(hh * dh, (hh + 1) * dh)
        q1h, q2h = q1[:, sl], q2[:, sl]
        k1h, k2h = k1[:, sl], k2[:, sl]
        v1h, v2h = v1[:, sl], v2[:, sl]
        l11 = (q1h * k1h).sum(axis=1, keepdims=True) * scale
        l12 = (q1h * k2h).sum(axis=1, keepdims=True) * scale
        l21 = (q2h * k1h).sum(axis=1, keepdims=True) * scale
        l22 = (q2h * k2h).sum(axis=1, keepdims=True) * scale
        m1 = jnp.maximum(l11, l12)
        e11 = jnp.exp(l11 - m1)
        e12 = jnp.exp(l12 - m1)
        s1 = e11 + e12
        m2 = jnp.maximum(l21, l22)
        e21 = jnp.exp(l21 - m2)
        e22 = jnp.exp(l22 - m2)
        s2 = e21 + e22
        o_chunks1.append((e11 * v1h + e12 * v2h) / s1)
        o_chunks2.append((e21 * v1h + e22 * v2h) / s2)
    o1 = jnp.concatenate(o_chunks1, axis=1)
    o2 = jnp.concatenate(o_chunks2, axis=1)
    af = mm(o1 + o2, wo_ref[...]) * 0.5 + bo_ref[...][None, :]

    w1 = w1_ref[...]
    h = (mm(se, w1[0:OUT, :]) + mm(de, w1[OUT:2 * OUT, :])
         + mm(af, w1[2 * OUT:3 * OUT, :]) + b1_ref[...][None, :])
    h = _gelu(h)
    h = h * (g1_ref[...] * _BN_SCALE)[None, :] + be1_ref[...][None, :]
    h = _gelu(mm(h, w2_ref[...]) + b2_ref[...][None, :])
    h = h * (g2_ref[...] * _BN_SCALE)[None, :] + be2_ref[...][None, :]
    h = _gelu(mm(h, w3_ref[...]) + b3_ref[...][None, :])
    h = h * (g3_ref[...] * _BN_SCALE)[None, :] + be3_ref[...][None, :]
    h = _gelu(mm(h, w4_ref[...]) + b4_ref[...][None, :])
    h = mm(h, w5_ref[...]) + b5_ref[...][None, :]
    out_ref[...] = jax.nn.sigmoid(h)


def _head(se, de, p):
    npred = se.shape[0]
    grid = npred // _HR
    row = pl.BlockSpec((_HR, OUT), lambda i: (i, 0))
    full = lambda a: pl.BlockSpec(a.shape, lambda i: (0,) * a.ndim)
    args = [
        p["Wq"], p["Wk"], p["Wv"], p["Wo"], p["bq"], p["bk"], p["bv"], p["bo"],
        p["Wl1"], p["bl1"], p["gl1"], p["bel1"],
        p["Wl2"], p["bl2"], p["gl2"], p["bel2"],
        p["Wl3"], p["bl3"], p["gl3"], p["bel3"],
        p["Wl4"], p["bl4"], p["Wl5"], p["bl5"],
    ]
    out = pl.pallas_call(
        _head_body,
        grid=(grid,),
        in_specs=[row, row] + [full(a) for a in args],
        out_specs=pl.BlockSpec((_HR, 1), lambda i: (i, 0)),
        out_shape=jax.ShapeDtypeStruct((npred, 1), jnp.float32),
    )(se, de, *args)
    return out[:, 0]


def kernel(x, params, edge_index, pred_edge_index):
    p = params
    n = x.shape[0]
    loop = jnp.arange(n, dtype=edge_index.dtype)
    src = jnp.concatenate([edge_index[0], loop])
    dst = jnp.concatenate([edge_index[1], loop])
    id1 = x @ p["Wr1"] + p["br1"]
    x1 = jax.nn.elu(_bn(_gat_conv(x, src, dst, p["W1"], p["as1"], p["ad1"], p["b1"], 8, True, n), p["g1"], p["be1"])) + id1
    x2 = jax.nn.elu(_bn(_gat_conv(x1, src, dst, p["W2"], p["as2"], p["ad2"], p["b2"], 8, True, n), p["g2"], p["be2"])) + x1
    x3 = jax.nn.elu(_bn(_gat_conv(x2, src, dst, p["W3"], p["as3"], p["ad3"], p["b3"], 4, True, n), p["g3"], p["be3"])) + x2
    x4 = jax.nn.elu(_bn(_gat_conv(x3, src, dst, p["W4"], p["as4"], p["ad4"], p["b4"], 4, True, n), p["g4"], p["be4"])) + x3
    id4 = x1 @ p["Wr4"] + p["br4"]
    x5 = _bn(_gat_conv(x4, src, dst, p["W5"], p["as5"], p["ad5"], p["b5"], 1, False, n), p["g5"], p["be5"]) + id4
    s, d = pred_edge_index[0], pred_edge_index[1]
    se = x5[s]
    de = x5[d]
    return _head(se, de, p)


# speedup vs baseline: 7.4146x; 7.2608x over previous
import functools

import jax
import jax.numpy as jnp
import numpy as np
from jax import lax
from jax.experimental import pallas as pl
from jax.experimental.pallas import tpu as pltpu
from jax.experimental.pallas import tpu_sc as plsc

N = 10000
NPAD = 10240
NW = 32
NB = NPAD // NW
E = 170000
EA = 5376
EDGE_ALLOC = EA * NW
RS_ALLOC = NPAD + 256

KA = 256
NCA = EA // KA

KB = 128

_BN_SCALE = np.float32(1.0 / np.sqrt(1.0 + 1e-5))

_MESH = dict(core_axis_name="c", subcore_axis_name="s", num_cores=2,
             num_subcores=16)


def _gelu(x):
    return 0.5 * x * (1.0 + jax.lax.erf(x * np.float32(1.0 / np.sqrt(2.0))))


def _elu(x):
    return jnp.where(x > 0.0, x, jnp.exp(x) - 1.0)


def _mm(a, w):
    return jax.lax.dot_general(a, w, (((1,), (0,)), ((), ())),
                               preferred_element_type=jnp.float32)


@functools.cache
def _make_phase_a():
    mesh = plsc.VectorSubcoreMesh(**_MESH)

    @functools.partial(
        pl.kernel,
        out_type=jax.ShapeDtypeStruct((EDGE_ALLOC, 16), jnp.float32),
        mesh=mesh,
        scratch_types=[
            pltpu.VMEM((KA,), jnp.int32),
            pltpu.VMEM((KA,), jnp.int32),
            pltpu.VMEM((KA, 128), jnp.float32),
            pltpu.VMEM((KA, 128), jnp.float32),
            pltpu.VMEM((KA, 16), jnp.float32),
            pltpu.VMEM((16, 16), jnp.float32),
            pltpu.SemaphoreType.DMA,
            pltpu.SemaphoreType.DMA,
        ],
    )
    def phase_a(as_hbm, ad_hbm, srcs_hbm, dsts_hbm, m_hbm, f_hbm,
                sbuf, dbuf, asbuf, adbuf, fbuf, mbuf, sem1, sem2):
        w = lax.axis_index("s") * 2 + lax.axis_index("c")
        pltpu.sync_copy(m_hbm, mbuf)
        mvec = mbuf[0]

        def chunk(ci, _):
            cb = w * EA + ci * KA
            pltpu.sync_copy(srcs_hbm.at[pl.ds(cb, KA)], sbuf)
            pltpu.sync_copy(dsts_hbm.at[pl.ds(cb, KA)], dbuf)
            cp1 = pltpu.async_copy(as_hbm.at[sbuf], asbuf, sem1)
            cp2 = pltpu.async_copy(ad_hbm.at[dbuf], adbuf, sem2)
            cp1.wait()
            cp2.wait()

            def ebody(el, _):
                t = asbuf[el, pl.ds(0, 16)] + adbuf[el, pl.ds(0, 16)]
                t = jnp.where(t >= 0.0, t, t * 0.2)
                fbuf[el] = jnp.exp(t - mvec)
                return 0
            lax.fori_loop(0, KA, ebody, 0)
            pltpu.sync_copy(fbuf, f_hbm.at[pl.ds(cb, KA)])
            return 0
        lax.fori_loop(0, NCA, chunk, 0)

    return phase_a


@functools.cache
def _make_phase_b(c, HH, hb):
    nj = 384 // 16
    head_of_j = [(j * 16) // c for j in range(nj)]
    mesh = plsc.VectorSubcoreMesh(**_MESH)

    @functools.partial(
        pl.kernel,
        out_type=jax.ShapeDtypeStruct((NPAD, 384), jnp.float32),
        mesh=mesh,
        scratch_types=[
            pltpu.VMEM((KB,), jnp.int32),
            pltpu.VMEM((KB + 16,), jnp.int32),
            pltpu.VMEM((KB, 384), jnp.float32),
            pltpu.VMEM((KB, 16), jnp.float32),
            pltpu.VMEM((64, 384), jnp.float32),
            pltpu.VMEM((64, 16), jnp.float32),
            pltpu.VMEM((512,), jnp.int32),
            pltpu.SemaphoreType.DMA,
        ],
    )
    def phase_b(h_hbm, srcs_hbm, dsts_hbm, f_hbm, rs_hbm, out_hbm,
                sbuf, dvm, rowbuf, fvm, accbuf, denbuf, rs_vm, sem):
        cidx = lax.axis_index("c")
        sidx = lax.axis_index("s")
        w = sidx * 2 + cidx
        nb0 = w * NB
        rb = (nb0 // 128) * 128
        off = nb0 - rb
        pltpu.sync_copy(rs_hbm.at[pl.ds(rb, 512)], rs_vm)
        zero = jnp.zeros((16,), jnp.float32)

        def subblock(sb, _):
            blk0 = nb0 + sb * 64

            def zrow(r, _):
                for j in range(nj):
                    accbuf[r, pl.ds(j * 16, 16)] = zero
                denbuf[r] = zero
                return 0
            lax.fori_loop(0, 64, zrow, 0)

            e0 = rs_vm[pl.ds(off + sb * 64, 16)][0]
            e1 = rs_vm[pl.ds(off + sb * 64 + 64, 16)][0]
            base = (e0 // KB) * KB
            nchunks = (e1 - base + KB - 1) // KB

            def chunk(ci, _):
                cb = base + ci * KB
                pltpu.sync_copy(srcs_hbm.at[pl.ds(cb, KB)], sbuf)
                pltpu.sync_copy(dsts_hbm.at[pl.ds(cb, KB)],
                                dvm.at[pl.ds(0, KB)])
                pltpu.sync_copy(f_hbm.at[pl.ds(cb, KB)], fvm)
                pltpu.async_copy(h_hbm.at[sbuf], rowbuf, sem).wait()
                es = jnp.maximum(cb, e0)
                ee = jnp.minimum(cb + KB, e1)

                def edge(e, _):
                    el = e - cb
                    r = dvm[pl.ds(el, 16)][0] - blk0
                    frow = fvm[el]
                    denbuf[r] = denbuf[r] + frow
                    fsp = [jnp.full((16,), frow[hb + hh])
                           for hh in range(HH)]
                    for j in range(nj):
                        accbuf[r, pl.ds(j * 16, 16)] = (
                            accbuf[r, pl.ds(j * 16, 16)]
                            + fsp[head_of_j[j]]
                            * rowbuf[el, pl.ds(j * 16, 16)])
                    return 0
                lax.fori_loop(es, ee, edge, 0)
                return 0
            lax.fori_loop(0, nchunks, chunk, 0)

            def norm(r, _):
                drow = denbuf[r]
                recips = []
                for hh in range(HH):
                    densp = jnp.full((16,), drow[hb + hh])
                    recips.append(jnp.where(densp > 0.0, 1.0 / densp, zero))
                for j in range(nj):
                    accbuf[r, pl.ds(j * 16, 16)] = (
                        accbuf[r, pl.ds(j * 16, 16)] * recips[head_of_j[j]])
                return 0
            lax.fori_loop(0, 64, norm, 0)
            pltpu.sync_copy(accbuf, out_hbm.at[pl.ds(blk0, 64)])
            return 0
        lax.fori_loop(0, NB // 64, subblock, 0)

    return phase_b


GROWS_TOT = 100352
GROWS = GROWS_TOT // NW
GK = 224
NCG = GROWS // GK


@functools.cache
def _make_gather():
    mesh = plsc.VectorSubcoreMesh(**_MESH)

    @functools.partial(
        pl.kernel,
        out_type=jax.ShapeDtypeStruct((GROWS_TOT, 384), jnp.float32),
        mesh=mesh,
        scratch_types=[
            pltpu.VMEM((GK,), jnp.int32),
            pltpu.VMEM((GK, 384), jnp.float32),
            pltpu.SemaphoreType.DMA,
        ],
    )
    def gather_rows(x5_hbm, idx_hbm, out_hbm, gi, gr, sem):
        w = lax.axis_index("s") * 2 + lax.axis_index("c")
        r0 = w * GROWS

        def chunk(ci, _):
            off = r0 + ci * GK
            pltpu.sync_copy(idx_hbm.at[pl.ds(off, GK)], gi)
            pltpu.async_copy(x5_hbm.at[gi], gr, sem).wait()
            pltpu.sync_copy(gr, out_hbm.at[pl.ds(off, GK)])
            return 0
        lax.fori_loop(0, NCG, chunk, 0)

    return gather_rows


_R = 1000


def _attn_cols(h, a_ref, H, c):
    cols = []
    for i in range(H):
        arow = a_ref[i:i + 1, :]
        cols.append((h[:, i * c:(i + 1) * c] * arow).sum(axis=1, keepdims=True))
    cols.append(jnp.zeros((h.shape[0], 128 - H), jnp.float32))
    return jnp.concatenate(cols, axis=1)


def _row_spec(width):
    return pl.BlockSpec((_R, width), lambda i: (i, 0))


def _full_spec(a):
    return pl.BlockSpec(a.shape, lambda i: (0,) * a.ndim)


def _tc_call(body, row_ins, full_ins, out_widths):
    specs = [_row_spec(a.shape[1]) for a in row_ins]
    specs += [_full_spec(a) for a in full_ins]
    outs = [jax.ShapeDtypeStruct((N, wdt), jnp.float32) for wdt in out_widths]
    return pl.pallas_call(
        body,
        grid=(N // _R,),
        in_specs=specs,
        out_specs=[_row_spec(wdt) for wdt in out_widths],
        out_shape=outs,
    )(*row_ins, *full_ins)


def _mm_first(x, W1, Wr1, br1, a_s, a_d):
    H, c = 8, 96

    def body(x_ref, w_ref, wr_ref, brr, as_ref, ad_ref,
             hlo_ref, hhi_ref, aso_ref, ado_ref, id_ref):
        xb = x_ref[...]
        h = _mm(xb, w_ref[...])
        hlo_ref[...] = h[:, :384]
        hhi_ref[...] = h[:, 384:]
        aso_ref[...] = _attn_cols(h, as_ref, H, c)
        ado_ref[...] = _attn_cols(h, ad_ref, H, c)
        id_ref[...] = _mm(xb, wr_ref[...]) + brr[...][None, :]

    return _tc_call(body, [x], [W1, Wr1, br1, a_s, a_d],
                    [384, 384, 128, 128, 768])


def _post(agg_lo, agg_hi, b, g, be, res):
    agg = jnp.concatenate([agg_lo, agg_hi], axis=1)
    v = agg + b[...][None, :]
    return _elu((g[...] * _BN_SCALE)[None, :] * v + be[...][None, :]) + res


def _mm_mid(agg_lo, agg_hi, b, g, be, res, W, a_s, a_d, H, c,
            Wr4=None, br4=None):
    extra = Wr4 is not None
    widths = [768, 384, 384, 128, 128]
    if extra:
        widths.append(384)

    def body2(alo, ahi, brr, grr, berr, resr, *rest):
        if extra:
            w_ref, as_ref, ad_ref, wr4, br4r = rest[:5]
            outs = rest[5:]
        else:
            w_ref, as_ref, ad_ref = rest[:3]
            outs = rest[3:]
        y = _post(alo[...], ahi[...], brr, grr, berr, resr[...])
        h = _mm(y, w_ref[...])
        outs[0][...] = y
        outs[1][...] = h[:, :384]
        outs[2][...] = h[:, 384:]
        outs[3][...] = _attn_cols(h, as_ref, H, c)
        outs[4][...] = _attn_cols(h, ad_ref, H, c)
        if extra:
            outs[5][...] = _mm(y, wr4[...]) + br4r[...][None, :]

    return pl.pallas_call(
            body2,
            grid=(N // _R,),
            in_specs=[_row_spec(384), _row_spec(384), _full_spec(b),
                      _full_spec(g), _full_spec(be), _row_spec(768)]
                     + [_full_spec(a) for a in ([W, a_s, a_d, Wr4, br4]
                                                if extra else [W, a_s, a_d])],
            out_specs=[_row_spec(wdt) for wdt in widths],
            out_shape=[jax.ShapeDtypeStruct((N, wdt), jnp.float32)
                       for wdt in widths],
        )(agg_lo, agg_hi, b, g, be, res,
          *([W, a_s, a_d, Wr4, br4] if extra else [W, a_s, a_d]))


def _mm_last(agg_lo, agg_hi, b, g, be, res, W, a_s, a_d):
    H, c = 1, 384

    def body(alo, ahi, brr, grr, berr, resr, w_ref, as_ref, ad_ref,
             h_ref, aso_ref, ado_ref):
        y = _post(alo[...], ahi[...], brr, grr, berr, resr[...])
        h = _mm(y, w_ref[...])
        h_ref[...] = h
        aso_ref[...] = _attn_cols(h, as_ref, H, c)
        ado_ref[...] = _attn_cols(h, ad_ref, H, c)

    return pl.pallas_call(
        body,
        grid=(N // _R,),
        in_specs=[_row_spec(384), _row_spec(384), _full_spec(b),
                  _full_spec(g), _full_spec(be), _row_spec(768),
                  _full_spec(W), _full_spec(a_s), _full_spec(a_d)],
        out_specs=[_row_spec(384), _row_spec(128), _row_spec(128)],
        out_shape=[jax.ShapeDtypeStruct((N, 384), jnp.float32),
                   jax.ShapeDtypeStruct((N, 128), jnp.float32),
                   jax.ShapeDtypeStruct((N, 128), jnp.float32)],
    )(agg_lo, agg_hi, b, g, be, res, W, a_s, a_d)


def _post5(agg5, b, g, be, id4):
    def body(aref, brr, grr, berr, idr, out_ref):
        v = aref[...] + brr[...][None, :]
        out_ref[...] = ((grr[...] * _BN_SCALE)[None, :] * v
                        + berr[...][None, :] + idr[...])

    return pl.pallas_call(
        body,
        grid=(N // _R,),
        in_specs=[_row_spec(384), _full_spec(b), _full_spec(g),
                  _full_spec(be), _row_spec(384)],
        out_specs=_row_spec(384),
        out_shape=jax.ShapeDtypeStruct((N, 384), jnp.float32),
    )(agg5, b, g, be, id4)


_HR = 1568
OUT = 384


def _head_body(se_ref, de_ref,
               wq_ref, wk_ref, wv_ref, wo_ref, bq_ref, bk_ref, bv_ref, bo_ref,
               w1_ref, b1_ref, g1_ref, be1_ref,
               w2_ref, b2_ref, g2_ref, be2_ref,
               w3_ref, b3_ref, g3_ref, be3_ref,
               w4_ref, b4_ref, w5_ref, b5_ref,
               out_ref):
    se = se_ref[...]
    de = de_ref[...]

    q1 = _mm(se, wq_ref[...]) + bq_ref[...][None, :]
    q2 = _mm(de, wq_ref[...]) + bq_ref[...][None, :]
    k1 = _mm(se, wk_ref[...]) + bk_ref[...][None, :]
    k2 = _mm(de, wk_ref[...]) + bk_ref[...][None, :]
    v1 = _mm(se, wv_ref[...]) + bv_ref[...][None, :]
    v2 = _mm(de, wv_ref[...]) + bv_ref[...][None, :]

    dh = 96
    scale = 1.0 / np.sqrt(dh)
    o_chunks1 = []
    o_chunks2 = []
    for hh in range(4):
        sl = slice(hh * dh, (hh + 1) * dh)
        q1h, q2h = q1[:, sl], q2[:, sl]
        k1h, k2h = k1[:, sl], k2[:, sl]
        v1h, v2h = v1[:, sl], v2[:, sl]
        l11 = (q1h * k1h).sum(axis=1, keepdims=True) * scale
        l12 = (q1h * k2h).sum(axis=1, keepdims=True) * scale
        l21 = (q2h * k1h).sum(axis=1, keepdims=True) * scale
        l22 = (q2h * k2h).sum(axis=1, keepdims=True) * scale
        m1 = jnp.maximum(l11, l12)
        e11 = jnp.exp(l11 - m1)
        e12 = jnp.exp(l12 - m1)
        s1 = e11 + e12
        m2 = jnp.maximum(l21, l22)
        e21 = jnp.exp(l21 - m2)
        e22 = jnp.exp(l22 - m2)
        s2 = e21 + e22
        o_chunks1.append((e11 * v1h + e12 * v2h) / s1)
        o_chunks2.append((e21 * v1h + e22 * v2h) / s2)
    o1 = jnp.concatenate(o_chunks1, axis=1)
    o2 = jnp.concatenate(o_chunks2, axis=1)
    af = _mm(o1 + o2, wo_ref[...]) * 0.5 + bo_ref[...][None, :]

    w1 = w1_ref[...]
    h = (_mm(se, w1[0:OUT, :]) + _mm(de, w1[OUT:2 * OUT, :])
         + _mm(af, w1[2 * OUT:3 * OUT, :]) + b1_ref[...][None, :])
    h = _gelu(h)
    h = h * (g1_ref[...] * _BN_SCALE)[None, :] + be1_ref[...][None, :]
    h = _gelu(_mm(h, w2_ref[...]) + b2_ref[...][None, :])
    h = h * (g2_ref[...] * _BN_SCALE)[None, :] + be2_ref[...][None, :]
    h = _gelu(_mm(h, w3_ref[...]) + b3_ref[...][None, :])
    h = h * (g3_ref[...] * _BN_SCALE)[None, :] + be3_ref[...][None, :]
    h = _gelu(_mm(h, w4_ref[...]) + b4_ref[...][None, :])
    h = _mm(h, w5_ref[...]) + b5_ref[...][None, :]
    out_ref[...] = jax.nn.sigmoid(h)


def _head(G, p):
    nblk = GROWS_TOT // 2 // _HR
    se_spec = pl.BlockSpec((_HR, OUT), lambda i: (i, 0))
    de_spec = pl.BlockSpec((_HR, OUT), lambda i: (i + nblk, 0))
    args = [
        p["Wq"], p["Wk"], p["Wv"], p["Wo"], p["bq"], p["bk"], p["bv"], p["bo"],
        p["Wl1"], p["bl1"], p["gl1"], p["bel1"],
        p["Wl2"], p["bl2"], p["gl2"], p["bel2"],
        p["Wl3"], p["bl3"], p["gl3"], p["bel3"],
        p["Wl4"], p["bl4"], p["Wl5"], p["bl5"],
    ]
    out = pl.pallas_call(
        _head_body,
        grid=(nblk,),
        in_specs=[se_spec, de_spec] + [_full_spec(a) for a in args],
        out_specs=pl.BlockSpec((_HR, 1), lambda i: (i, 0)),
        out_shape=jax.ShapeDtypeStruct((GROWS_TOT // 2, 1), jnp.float32),
    )(G, G, *args)
    return out[:, 0]


def _m_arr(a_s, a_d, H):
    M = jnp.max(a_s[:, :H], axis=0) + jnp.max(a_d[:, :H], axis=0)
    M = jnp.maximum(M, 0.2 * M)
    mpad = jnp.pad(M, (0, 16 - H))
    return jnp.tile(mpad[None, :], (16, 1))


def _gat_layer(h_lo, h_hi, a_s, a_d, H, c, srcs_p, dsts_p, rs):
    HH = 384 // c
    f = _make_phase_a()(a_s, a_d, srcs_p, dsts_p, _m_arr(a_s, a_d, H))
    agg_lo = _make_phase_b(c, HH, 0)(h_lo, srcs_p, dsts_p, f, rs)
    if h_hi is None:
        return agg_lo, None
    agg_hi = _make_phase_b(c, HH, HH)(h_hi, srcs_p, dsts_p, f, rs)
    return agg_lo, agg_hi


def kernel(x, params, edge_index, pred_edge_index):
    p = params
    loop = jnp.arange(N, dtype=jnp.int32)
    src = jnp.concatenate([edge_index[0], loop])
    dst = jnp.concatenate([edge_index[1], loop])
    order = jnp.argsort(dst)
    dsts = dst[order]
    srcs = src[order]
    rs = jnp.searchsorted(
        dsts, jnp.arange(RS_ALLOC, dtype=jnp.int32)).astype(jnp.int32)
    pad = jnp.zeros((EDGE_ALLOC - E,), jnp.int32)
    srcs_p = jnp.concatenate([srcs, pad])
    dsts_p = jnp.concatenate([dsts, pad])

    h_lo, h_hi, as1, ad1, id1 = _mm_first(
        x, p["W1"], p["Wr1"], p["br1"], p["as1"], p["ad1"])
    agg_lo, agg_hi = _gat_layer(h_lo, h_hi, as1, ad1, 8, 96,
                                srcs_p, dsts_p, rs)

    y1, h_lo, h_hi, as2, ad2, id4 = _mm_mid(
        agg_lo, agg_hi, p["b1"], p["g1"], p["be1"], id1, p["W2"],
        p["as2"], p["ad2"], 8, 96, p["Wr4"], p["br4"])
    agg_lo, agg_hi = _gat_layer(h_lo, h_hi, as2, ad2, 8, 96,
                                srcs_p, dsts_p, rs)

    y2, h_lo, h_hi, as3, ad3 = _mm_mid(
        agg_lo, agg_hi, p["b2"], p["g2"], p["be2"], y1, p["W3"],
        p["as3"], p["ad3"], 4, 192)
    agg_lo, agg_hi = _gat_layer(h_lo, h_hi, as3, ad3, 4, 192,
                                srcs_p, dsts_p, rs)

    y3, h_lo, h_hi, as4, ad4 = _mm_mid(
        agg_lo, agg_hi, p["b3"], p["g3"], p["be3"], y2, p["W4"],
        p["as4"], p["ad4"], 4, 192)
    agg_lo, agg_hi = _gat_layer(h_lo, h_hi, as4, ad4, 4, 192,
                                srcs_p, dsts_p, rs)

    h5, as5, ad5 = _mm_last(
        agg_lo, agg_hi, p["b4"], p["g4"], p["be4"], y3, p["W5"],
        p["as5"], p["ad5"])
    agg5, _ = _gat_layer(h5, None, as5, ad5, 1, 384, srcs_p, dsts_p, rs)

    x5 = _post5(agg5, p["b5"], p["g5"], p["be5"], id4)

    npred = pred_edge_index.shape[1]
    half = GROWS_TOT // 2
    ipad = jnp.zeros((half - npred,), jnp.int32)
    idx_all = jnp.concatenate(
        [pred_edge_index[0], ipad, pred_edge_index[1], ipad])
    G = _make_gather()(x5, idx_all)
    out = _head(G, p)
    return out[:npred]


# speedup vs baseline: 14.1809x; 1.9126x over previous
import functools

import jax
import jax.numpy as jnp
import numpy as np
from jax import lax
from jax.experimental import pallas as pl
from jax.experimental.pallas import tpu as pltpu
from jax.experimental.pallas import tpu_sc as plsc

N = 10000
NPAD = 10240
NW = 32
NB = NPAD // NW
E = 170000
EA = 5376
EDGE_ALLOC = EA * NW
RS_ALLOC = NPAD + 256

KA = 256
NCA = EA // KA

KB = 128

_BN_SCALE = np.float32(1.0 / np.sqrt(1.0 + 1e-5))

_MESH = dict(core_axis_name="c", subcore_axis_name="s", num_cores=2,
             num_subcores=16)


def _gelu(x):
    return 0.5 * x * (1.0 + jax.lax.erf(x * np.float32(1.0 / np.sqrt(2.0))))


def _elu(x):
    return jnp.where(x > 0.0, x, jnp.exp(x) - 1.0)


def _mm(a, w):
    return jax.lax.dot_general(a, w, (((1,), (0,)), ((), ())),
                               preferred_element_type=jnp.float32)


@functools.cache
def _make_phase_a():
    mesh = plsc.VectorSubcoreMesh(**_MESH)

    @functools.partial(
        pl.kernel,
        out_type=jax.ShapeDtypeStruct((EDGE_ALLOC, 16), jnp.float32),
        mesh=mesh,
        scratch_types=[
            pltpu.VMEM((KA,), jnp.int32),
            pltpu.VMEM((KA,), jnp.int32),
            pltpu.VMEM((KA, 128), jnp.float32),
            pltpu.VMEM((KA, 128), jnp.float32),
            pltpu.VMEM((KA, 16), jnp.float32),
            pltpu.VMEM((16, 16), jnp.float32),
            pltpu.SemaphoreType.DMA,
            pltpu.SemaphoreType.DMA,
        ],
    )
    def phase_a(as_hbm, ad_hbm, srcs_hbm, dsts_hbm, m_hbm, f_hbm,
                sbuf, dbuf, asbuf, adbuf, fbuf, mbuf, sem1, sem2):
        w = lax.axis_index("s") * 2 + lax.axis_index("c")
        pltpu.sync_copy(m_hbm, mbuf)
        mvec = mbuf[0]

        def chunk(ci, _):
            cb = w * EA + ci * KA
            pltpu.sync_copy(srcs_hbm.at[pl.ds(cb, KA)], sbuf)
            pltpu.sync_copy(dsts_hbm.at[pl.ds(cb, KA)], dbuf)
            cp1 = pltpu.async_copy(as_hbm.at[sbuf], asbuf, sem1)
            cp2 = pltpu.async_copy(ad_hbm.at[dbuf], adbuf, sem2)
            cp1.wait()
            cp2.wait()

            def ebody(el, _):
                t = asbuf[el, pl.ds(0, 16)] + adbuf[el, pl.ds(0, 16)]
                t = jnp.where(t >= 0.0, t, t * 0.2)
                fbuf[el] = jnp.exp(t - mvec)
                return 0
            lax.fori_loop(0, KA, ebody, 0)
            pltpu.sync_copy(fbuf, f_hbm.at[pl.ds(cb, KA)])
            return 0
        lax.fori_loop(0, NCA, chunk, 0)

    return phase_a


@functools.cache
def _make_phase_b(c, HH, hb):
    nj = 384 // 16
    head_of_j = [(j * 16) // c for j in range(nj)]
    mesh = plsc.VectorSubcoreMesh(**_MESH)

    @functools.partial(
        pl.kernel,
        out_type=jax.ShapeDtypeStruct((NPAD, 384), jnp.float32),
        mesh=mesh,
        scratch_types=[
            pltpu.VMEM((KB,), jnp.int32),
            pltpu.VMEM((KB + 16,), jnp.int32),
            pltpu.VMEM((KB, 384), jnp.float32),
            pltpu.VMEM((KB, 16), jnp.float32),
            pltpu.VMEM((64, 384), jnp.float32),
            pltpu.VMEM((64, 16), jnp.float32),
            pltpu.VMEM((512,), jnp.int32),
            pltpu.SemaphoreType.DMA,
        ],
    )
    def phase_b(h_hbm, srcs_hbm, dsts_hbm, f_hbm, rs_hbm, out_hbm,
                sbuf, dvm, rowbuf, fvm, accbuf, denbuf, rs_vm, sem):
        cidx = lax.axis_index("c")
        sidx = lax.axis_index("s")
        w = sidx * 2 + cidx
        nb0 = w * NB
        rb = (nb0 // 128) * 128
        off = nb0 - rb
        pltpu.sync_copy(rs_hbm.at[pl.ds(rb, 512)], rs_vm)
        zero = jnp.zeros((16,), jnp.float32)

        def subblock(sb, _):
            blk0 = nb0 + sb * 64

            def zrow(r, _):
                for j in range(nj):
                    accbuf[r, pl.ds(j * 16, 16)] = zero
                denbuf[r] = zero
                return 0
            lax.fori_loop(0, 64, zrow, 0)

            e0 = rs_vm[pl.ds(off + sb * 64, 16)][0]
            e1 = rs_vm[pl.ds(off + sb * 64 + 64, 16)][0]
            base = (e0 // KB) * KB
            nchunks = (e1 - base + KB - 1) // KB

            def chunk(ci, _):
                cb = base + ci * KB
                pltpu.sync_copy(srcs_hbm.at[pl.ds(cb, KB)], sbuf)
                pltpu.sync_copy(f_hbm.at[pl.ds(cb, KB)], fvm)
                pltpu.async_copy(h_hbm.at[sbuf], rowbuf, sem).wait()
                es = jnp.maximum(cb, e0)
                ee = jnp.minimum(cb + KB, e1)

                def node(r, relo):
                    rehi = rs_vm[pl.ds(off + sb * 64 + r + 1, 16)][0]
                    lo = jnp.maximum(relo, es)
                    hi = jnp.minimum(rehi, ee)

                    def edge(e, acc):
                        el = e - cb
                        frow = fvm[el]
                        fsp = [jnp.full((16,), frow[hb + hh])
                               for hh in range(HH)]
                        naccs = tuple(
                            acc[j] + fsp[head_of_j[j]]
                            * rowbuf[el, pl.ds(j * 16, 16)]
                            for j in range(nj))
                        return naccs + (acc[nj] + frow,)

                    accf = lax.fori_loop(lo, hi, edge,
                                         (zero,) * nj + (zero,))

                    @pl.when(hi > lo)
                    def _():
                        for j in range(nj):
                            accbuf[r, pl.ds(j * 16, 16)] = (
                                accbuf[r, pl.ds(j * 16, 16)] + accf[j])
                        denbuf[r] = denbuf[r] + accf[nj]
                    return rehi

                lax.fori_loop(0, 64, node,
                              rs_vm[pl.ds(off + sb * 64, 16)][0])
                return 0
            lax.fori_loop(0, nchunks, chunk, 0)

            def norm(r, _):
                drow = denbuf[r]
                recips = []
                for hh in range(HH):
                    densp = jnp.full((16,), drow[hb + hh])
                    recips.append(jnp.where(densp > 0.0, 1.0 / densp, zero))
                for j in range(nj):
                    accbuf[r, pl.ds(j * 16, 16)] = (
                        accbuf[r, pl.ds(j * 16, 16)] * recips[head_of_j[j]])
                return 0
            lax.fori_loop(0, 64, norm, 0)
            pltpu.sync_copy(accbuf, out_hbm.at[pl.ds(blk0, 64)])
            return 0
        lax.fori_loop(0, NB // 64, subblock, 0)

    return phase_b


GROWS_TOT = 100352
GROWS = GROWS_TOT // NW
GK = 224
NCG = GROWS // GK


@functools.cache
def _make_gather():
    mesh = plsc.VectorSubcoreMesh(**_MESH)

    @functools.partial(
        pl.kernel,
        out_type=jax.ShapeDtypeStruct((GROWS_TOT, 384), jnp.float32),
        mesh=mesh,
        scratch_types=[
            pltpu.VMEM((GK,), jnp.int32),
            pltpu.VMEM((GK, 384), jnp.float32),
            pltpu.SemaphoreType.DMA,
        ],
    )
    def gather_rows(x5_hbm, idx_hbm, out_hbm, gi, gr, sem):
        w = lax.axis_index("s") * 2 + lax.axis_index("c")
        r0 = w * GROWS

        def chunk(ci, _):
            off = r0 + ci * GK
            pltpu.sync_copy(idx_hbm.at[pl.ds(off, GK)], gi)
            pltpu.async_copy(x5_hbm.at[gi], gr, sem).wait()
            pltpu.sync_copy(gr, out_hbm.at[pl.ds(off, GK)])
            return 0
        lax.fori_loop(0, NCG, chunk, 0)

    return gather_rows


_R = 1000


def _attn_cols(h, a_ref, H, c):
    cols = []
    for i in range(H):
        arow = a_ref[i:i + 1, :]
        cols.append((h[:, i * c:(i + 1) * c] * arow).sum(axis=1, keepdims=True))
    cols.append(jnp.zeros((h.shape[0], 128 - H), jnp.float32))
    return jnp.concatenate(cols, axis=1)


def _row_spec(width):
    return pl.BlockSpec((_R, width), lambda i: (i, 0))


def _full_spec(a):
    return pl.BlockSpec(a.shape, lambda i: (0,) * a.ndim)


def _tc_call(body, row_ins, full_ins, out_widths):
    specs = [_row_spec(a.shape[1]) for a in row_ins]
    specs += [_full_spec(a) for a in full_ins]
    outs = [jax.ShapeDtypeStruct((N, wdt), jnp.float32) for wdt in out_widths]
    return pl.pallas_call(
        body,
        grid=(N // _R,),
        in_specs=specs,
        out_specs=[_row_spec(wdt) for wdt in out_widths],
        out_shape=outs,
    )(*row_ins, *full_ins)


def _mm_first(x, W1, Wr1, br1, a_s, a_d):
    H, c = 8, 96

    def body(x_ref, w_ref, wr_ref, brr, as_ref, ad_ref,
             hlo_ref, hhi_ref, aso_ref, ado_ref, id_ref):
        xb = x_ref[...]
        h = _mm(xb, w_ref[...])
        hlo_ref[...] = h[:, :384]
        hhi_ref[...] = h[:, 384:]
        aso_ref[...] = _attn_cols(h, as_ref, H, c)
        ado_ref[...] = _attn_cols(h, ad_ref, H, c)
        id_ref[...] = _mm(xb, wr_ref[...]) + brr[...][None, :]

    return _tc_call(body, [x], [W1, Wr1, br1, a_s, a_d],
                    [384, 384, 128, 128, 768])


def _post(agg_lo, agg_hi, b, g, be, res):
    agg = jnp.concatenate([agg_lo, agg_hi], axis=1)
    v = agg + b[...][None, :]
    return _elu((g[...] * _BN_SCALE)[None, :] * v + be[...][None, :]) + res


def _mm_mid(agg_lo, agg_hi, b, g, be, res, W, a_s, a_d, H, c,
            Wr4=None, br4=None):
    extra = Wr4 is not None
    widths = [768, 384, 384, 128, 128]
    if extra:
        widths.append(384)

    def body2(alo, ahi, brr, grr, berr, resr, *rest):
        if extra:
            w_ref, as_ref, ad_ref, wr4, br4r = rest[:5]
            outs = rest[5:]
        else:
            w_ref, as_ref, ad_ref = rest[:3]
            outs = rest[3:]
        y = _post(alo[...], ahi[...], brr, grr, berr, resr[...])
        h = _mm(y, w_ref[...])
        outs[0][...] = y
        outs[1][...] = h[:, :384]
        outs[2][...] = h[:, 384:]
        outs[3][...] = _attn_cols(h, as_ref, H, c)
        outs[4][...] = _attn_cols(h, ad_ref, H, c)
        if extra:
            outs[5][...] = _mm(y, wr4[...]) + br4r[...][None, :]

    return pl.pallas_call(
            body2,
            grid=(N // _R,),
            in_specs=[_row_spec(384), _row_spec(384), _full_spec(b),
                      _full_spec(g), _full_spec(be), _row_spec(768)]
                     + [_full_spec(a) for a in ([W, a_s, a_d, Wr4, br4]
                                                if extra else [W, a_s, a_d])],
            out_specs=[_row_spec(wdt) for wdt in widths],
            out_shape=[jax.ShapeDtypeStruct((N, wdt), jnp.float32)
                       for wdt in widths],
        )(agg_lo, agg_hi, b, g, be, res,
          *([W, a_s, a_d, Wr4, br4] if extra else [W, a_s, a_d]))


def _mm_last(agg_lo, agg_hi, b, g, be, res, W, a_s, a_d):
    H, c = 1, 384

    def body(alo, ahi, brr, grr, berr, resr, w_ref, as_ref, ad_ref,
             h_ref, aso_ref, ado_ref):
        y = _post(alo[...], ahi[...], brr, grr, berr, resr[...])
        h = _mm(y, w_ref[...])
        h_ref[...] = h
        aso_ref[...] = _attn_cols(h, as_ref, H, c)
        ado_ref[...] = _attn_cols(h, ad_ref, H, c)

    return pl.pallas_call(
        body,
        grid=(N // _R,),
        in_specs=[_row_spec(384), _row_spec(384), _full_spec(b),
                  _full_spec(g), _full_spec(be), _row_spec(768),
                  _full_spec(W), _full_spec(a_s), _full_spec(a_d)],
        out_specs=[_row_spec(384), _row_spec(128), _row_spec(128)],
        out_shape=[jax.ShapeDtypeStruct((N, 384), jnp.float32),
                   jax.ShapeDtypeStruct((N, 128), jnp.float32),
                   jax.ShapeDtypeStruct((N, 128), jnp.float32)],
    )(agg_lo, agg_hi, b, g, be, res, W, a_s, a_d)


def _post5(agg5, b, g, be, id4):
    def body(aref, brr, grr, berr, idr, out_ref):
        v = aref[...] + brr[...][None, :]
        out_ref[...] = ((grr[...] * _BN_SCALE)[None, :] * v
                        + berr[...][None, :] + idr[...])

    return pl.pallas_call(
        body,
        grid=(N // _R,),
        in_specs=[_row_spec(384), _full_spec(b), _full_spec(g),
                  _full_spec(be), _row_spec(384)],
        out_specs=_row_spec(384),
        out_shape=jax.ShapeDtypeStruct((N, 384), jnp.float32),
    )(agg5, b, g, be, id4)


_HR = 1568
OUT = 384


def _head_body(se_ref, de_ref,
               wq_ref, wk_ref, wv_ref, wo_ref, bq_ref, bk_ref, bv_ref, bo_ref,
               w1_ref, b1_ref, g1_ref, be1_ref,
               w2_ref, b2_ref, g2_ref, be2_ref,
               w3_ref, b3_ref, g3_ref, be3_ref,
               w4_ref, b4_ref, w5_ref, b5_ref,
               out_ref):
    se = se_ref[...]
    de = de_ref[...]

    q1 = _mm(se, wq_ref[...]) + bq_ref[...][None, :]
    q2 = _mm(de, wq_ref[...]) + bq_ref[...][None, :]
    k1 = _mm(se, wk_ref[...]) + bk_ref[...][None, :]
    k2 = _mm(de, wk_ref[...]) + bk_ref[...][None, :]
    v1 = _mm(se, wv_ref[...]) + bv_ref[...][None, :]
    v2 = _mm(de, wv_ref[...]) + bv_ref[...][None, :]

    dh = 96
    scale = 1.0 / np.sqrt(dh)
    o_chunks1 = []
    o_chunks2 = []
    for hh in range(4):
        sl = slice(hh * dh, (hh + 1) * dh)
        q1h, q2h = q1[:, sl], q2[:, sl]
        k1h, k2h = k1[:, sl], k2[:, sl]
        v1h, v2h = v1[:, sl], v2[:, sl]
        l11 = (q1h * k1h).sum(axis=1, keepdims=True) * scale
        l12 = (q1h * k2h).sum(axis=1, keepdims=True) * scale
        l21 = (q2h * k1h).sum(axis=1, keepdims=True) * scale
        l22 = (q2h * k2h).sum(axis=1, keepdims=True) * scale
        m1 = jnp.maximum(l11, l12)
        e11 = jnp.exp(l11 - m1)
        e12 = jnp.exp(l12 - m1)
        s1 = e11 + e12
        m2 = jnp.maximum(l21, l22)
        e21 = jnp.exp(l21 - m2)
        e22 = jnp.exp(l22 - m2)
        s2 = e21 + e22
        o_chunks1.append((e11 * v1h + e12 * v2h) / s1)
        o_chunks2.append((e21 * v1h + e22 * v2h) / s2)
    o1 = jnp.concatenate(o_chunks1, axis=1)
    o2 = jnp.concatenate(o_chunks2, axis=1)
    af = _mm(o1 + o2, wo_ref[...]) * 0.5 + bo_ref[...][None, :]

    w1 = w1_ref[...]
    h = (_mm(se, w1[0:OUT, :]) + _mm(de, w1[OUT:2 * OUT, :])
         + _mm(af, w1[2 * OUT:3 * OUT, :]) + b1_ref[...][None, :])
    h = _gelu(h)
    h = h * (g1_ref[...] * _BN_SCALE)[None, :] + be1_ref[...][None, :]
    h = _gelu(_mm(h, w2_ref[...]) + b2_ref[...][None, :])
    h = h * (g2_ref[...] * _BN_SCALE)[None, :] + be2_ref[...][None, :]
    h = _gelu(_mm(h, w3_ref[...]) + b3_ref[...][None, :])
    h = h * (g3_ref[...] * _BN_SCALE)[None, :] + be3_ref[...][None, :]
    h = _gelu(_mm(h, w4_ref[...]) + b4_ref[...][None, :])
    h = _mm(h, w5_ref[...]) + b5_ref[...][None, :]
    out_ref[...] = jax.nn.sigmoid(h)


def _head(G, p):
    nblk = GROWS_TOT // 2 // _HR
    se_spec = pl.BlockSpec((_HR, OUT), lambda i: (i, 0))
    de_spec = pl.BlockSpec((_HR, OUT), lambda i: (i + nblk, 0))
    args = [
        p["Wq"], p["Wk"], p["Wv"], p["Wo"], p["bq"], p["bk"], p["bv"], p["bo"],
        p["Wl1"], p["bl1"], p["gl1"], p["bel1"],
        p["Wl2"], p["bl2"], p["gl2"], p["bel2"],
        p["Wl3"], p["bl3"], p["gl3"], p["bel3"],
        p["Wl4"], p["bl4"], p["Wl5"], p["bl5"],
    ]
    out = pl.pallas_call(
        _head_body,
        grid=(nblk,),
        in_specs=[se_spec, de_spec] + [_full_spec(a) for a in args],
        out_specs=pl.BlockSpec((_HR, 1), lambda i: (i, 0)),
        out_shape=jax.ShapeDtypeStruct((GROWS_TOT // 2, 1), jnp.float32),
    )(G, G, *args)
    return out[:, 0]


def _m_arr(a_s, a_d, H):
    M = jnp.max(a_s[:, :H], axis=0) + jnp.max(a_d[:, :H], axis=0)
    M = jnp.maximum(M, 0.2 * M)
    mpad = jnp.pad(M, (0, 16 - H))
    return jnp.tile(mpad[None, :], (16, 1))


def _gat_layer(h_lo, h_hi, a_s, a_d, H, c, srcs_p, dsts_p, rs):
    HH = 384 // c
    f = _make_phase_a()(a_s, a_d, srcs_p, dsts_p, _m_arr(a_s, a_d, H))
    agg_lo = _make_phase_b(c, HH, 0)(h_lo, srcs_p, dsts_p, f, rs)
    if h_hi is None:
        return agg_lo, None
    agg_hi = _make_phase_b(c, HH, HH)(h_hi, srcs_p, dsts_p, f, rs)
    return agg_lo, agg_hi


def kernel(x, params, edge_index, pred_edge_index):
    p = params
    loop = jnp.arange(N, dtype=jnp.int32)
    src = jnp.concatenate([edge_index[0], loop])
    dst = jnp.concatenate([edge_index[1], loop])
    order = jnp.argsort(dst)
    dsts = dst[order]
    srcs = src[order]
    rs = jnp.searchsorted(
        dsts, jnp.arange(RS_ALLOC, dtype=jnp.int32)).astype(jnp.int32)
    pad = jnp.zeros((EDGE_ALLOC - E,), jnp.int32)
    srcs_p = jnp.concatenate([srcs, pad])
    dsts_p = jnp.concatenate([dsts, pad])

    h_lo, h_hi, as1, ad1, id1 = _mm_first(
        x, p["W1"], p["Wr1"], p["br1"], p["as1"], p["ad1"])
    agg_lo, agg_hi = _gat_layer(h_lo, h_hi, as1, ad1, 8, 96,
                                srcs_p, dsts_p, rs)

    y1, h_lo, h_hi, as2, ad2, id4 = _mm_mid(
        agg_lo, agg_hi, p["b1"], p["g1"], p["be1"], id1, p["W2"],
        p["as2"], p["ad2"], 8, 96, p["Wr4"], p["br4"])
    agg_lo, agg_hi = _gat_layer(h_lo, h_hi, as2, ad2, 8, 96,
                                srcs_p, dsts_p, rs)

    y2, h_lo, h_hi, as3, ad3 = _mm_mid(
        agg_lo, agg_hi, p["b2"], p["g2"], p["be2"], y1, p["W3"],
        p["as3"], p["ad3"], 4, 192)
    agg_lo, agg_hi = _gat_layer(h_lo, h_hi, as3, ad3, 4, 192,
                                srcs_p, dsts_p, rs)

    y3, h_lo, h_hi, as4, ad4 = _mm_mid(
        agg_lo, agg_hi, p["b3"], p["g3"], p["be3"], y2, p["W4"],
        p["as4"], p["ad4"], 4, 192)
    agg_lo, agg_hi = _gat_layer(h_lo, h_hi, as4, ad4, 4, 192,
                                srcs_p, dsts_p, rs)

    h5, as5, ad5 = _mm_last(
        agg_lo, agg_hi, p["b4"], p["g4"], p["be4"], y3, p["W5"],
        p["as5"], p["ad5"])
    agg5, _ = _gat_layer(h5, None, as5, ad5, 1, 384, srcs_p, dsts_p, rs)

    x5 = _post5(agg5, p["b5"], p["g5"], p["be5"], id4)

    npred = pred_edge_index.shape[1]
    half = GROWS_TOT // 2
    ipad = jnp.zeros((half - npred,), jnp.int32)
    idx_all = jnp.concatenate(
        [pred_edge_index[0], ipad, pred_edge_index[1], ipad])
    G = _make_gather()(x5, idx_all)
    out = _head(G, p)
    return out[:npred]
